# baseline (device time: 198406 ns/iter reference)
import jax
import jax.numpy as jnp
from jax import lax
from jax.experimental import pallas as pl
from jax.experimental.pallas import tpu as pltpu

N_DEV = 16
NEG_INF = -1e30


def kernel(Q, K, V):
    b, s, h, d = Q.shape
    bh = b * h
    half = bh // 2
    scale = d ** -0.5

    def to_t(x):
        return x.astype(jnp.bfloat16).transpose(0, 2, 3, 1).reshape(bh, d, s)

    q = to_t(Q * scale)
    k = to_t(K)
    v = to_t(V)

    def body(q_ref, k_ref, v_ref, out_ref,
             krbuf, vrbuf, klbuf, vlbuf, l_ref,
             send_sems, recv_sems):
        me = lax.axis_index("i")
        left = lax.rem(me + N_DEV - 1, N_DEV)
        right = lax.rem(me + 1, N_DEV)

        barrier = pltpu.get_barrier_semaphore()
        pl.semaphore_signal(barrier, inc=1, device_id=(left,),
                            device_id_type=pl.DeviceIdType.MESH)
        pl.semaphore_signal(barrier, inc=1, device_id=(right,),
                            device_id_type=pl.DeviceIdType.MESH)
        pl.semaphore_wait(barrier, 2)

        l_ref[...] = jnp.zeros(l_ref.shape, jnp.float32)
        out_ref[...] = jnp.zeros(out_ref.shape, jnp.float32)

        def flash_update(j, kj, vj):
            qj = q_ref[j]
            sc = lax.dot_general(kj, qj, (((0,), (0,)), ((), ())),
                                 preferred_element_type=jnp.float32)
            p = jnp.exp(sc)
            l_ref[j] = l_ref[j] + jnp.sum(p, axis=0, keepdims=True)
            out_ref[j] = out_ref[j] + lax.dot_general(
                vj, p.astype(jnp.bfloat16), (((1,), (0,)), ((), ())),
                preferred_element_type=jnp.float32)

        def compute_own():
            def own_body(j, carry):
                flash_update(j, k_ref[j], v_ref[j])
                return carry
            lax.fori_loop(0, bh, own_body, 0)

        def compute_slot(t):
            def right_body(j, carry):
                flash_update(j, krbuf[t, j], vrbuf[t, j])
                return carry
            lax.fori_loop(0, half, right_body, 0)

            def left_body(j, carry):
                flash_update(half + j, klbuf[t, j], vlbuf[t, j])
                return carry
            lax.fori_loop(0, half, left_body, 0)

        gs = half // 4
        streams = []
        for g in range(4):
            for inp, buf, base, tgt in (
                    (k_ref, krbuf, 0, right), (k_ref, klbuf, half, left),
                    (v_ref, vrbuf, 0, right), (v_ref, vlbuf, half, left)):
                r0 = g * gs
                streams.append(
                    (len(streams), inp.at[base + r0:base + r0 + gs],
                     buf, r0, tgt))

        def make_desc(stream, hop):
            si, src0, buf, r0, tgt = stream
            src = src0 if hop == 0 else buf.at[hop - 1, r0:r0 + gs]
            return pltpu.make_async_remote_copy(
                src_ref=src, dst_ref=buf.at[hop, r0:r0 + gs],
                send_sem=send_sems.at[si, hop], recv_sem=recv_sems.at[si, hop],
                device_id=(tgt,), device_id_type=pl.DeviceIdType.MESH)

        all_descs = []
        prev = []
        for st in streams:
            rd = make_desc(st, 0)
            rd.start()
            prev.append(rd)
        all_descs += prev
        compute_own()
        for hop in range(1, N_DEV - 1):
            cur = []
            for st in streams:
                prev[st[0]].wait_recv()
                rd = make_desc(st, hop)
                rd.start()
                cur.append(rd)
            all_descs += cur
            compute_slot(hop - 1)
            prev = cur
        for rd in prev:
            rd.wait_recv()
        compute_slot(N_DEV - 2)
        for rd in all_descs:
            rd.wait_send()

        out_ref[...] = out_ref[...] / l_ref[...]

    out = pl.pallas_call(
        body,
        out_shape=jax.ShapeDtypeStruct((bh, d, s), jnp.float32),
        in_specs=[pl.BlockSpec(memory_space=pltpu.VMEM)] * 3,
        out_specs=pl.BlockSpec(memory_space=pltpu.VMEM),
        scratch_shapes=[
            pltpu.VMEM((N_DEV - 1, half, d, s), jnp.bfloat16),
            pltpu.VMEM((N_DEV - 1, half, d, s), jnp.bfloat16),
            pltpu.VMEM((N_DEV - 1, half, d, s), jnp.bfloat16),
            pltpu.VMEM((N_DEV - 1, half, d, s), jnp.bfloat16),
            pltpu.VMEM((bh, 1, s), jnp.float32),
            pltpu.SemaphoreType.DMA((16, N_DEV - 1)),
            pltpu.SemaphoreType.DMA((16, N_DEV - 1)),
        ],
        compiler_params=pltpu.CompilerParams(collective_id=0),
    )(q, k, v)

    return out.reshape(b, h, d, s).transpose(0, 3, 1, 2)
